# Initial kernel scaffold; baseline (speedup 1.0000x reference)
import jax
import jax.numpy as jnp
from jax.experimental import pallas as pl


def kernel(x, edge_index, batch, W1, b1, attl1, attr1, W2, b2, attl2, attr2, pW1, pb1, pW2, pb2):
    raise NotImplementedError("write your pallas kernel here")


# speedup vs baseline: 64.7606x; 64.7606x over previous
import functools

import jax
import jax.numpy as jnp
from jax import lax
from jax.experimental import pallas as pl
from jax.experimental.pallas import tpu as pltpu
from jax.experimental.pallas import tpu_sc as plsc

N = 10000
E = 640000
HC = 128
HALF = 64
NSUB = 16
B = 128
NBATCH = E // B
RT = 400
NPERS = 640
ACCN = NSUB * NPERS
EPS = 1e-16


def _prep_body(do_relu, x0_ref, x1_ref, wt_ref, b_ref, al_ref, ar_ref, g_ref, arr_ref):
    c = pl.program_id(0)
    xb = jnp.concatenate([x0_ref[...], x1_ref[...]], axis=1)
    if do_relu:
        xb = jnp.maximum(xb, 0.0)
    xl = jnp.dot(xb, wt_ref[...], preferred_element_type=jnp.float32) + b_ref[...]
    al = al_ref[...] * xl
    ar = ar_ref[...] * xl
    al_h = jnp.where(c == 0, al[:, :HALF], al[:, HALF:])
    xl_h = jnp.where(c == 0, xl[:, :HALF], xl[:, HALF:])
    ar_h = jnp.where(c == 0, ar[:, :HALF], ar[:, HALF:])
    g_ref[...] = jnp.concatenate([al_h, xl_h], axis=1)
    arr_ref[...] = ar_h


def _make_prep(do_relu):
    nb = N // RT
    return pl.pallas_call(
        functools.partial(_prep_body, do_relu),
        grid=(2, nb),
        in_specs=[
            pl.BlockSpec((RT, HALF), lambda c, i: (i, 0)),
            pl.BlockSpec((RT, HALF), lambda c, i: (i, 0)),
            pl.BlockSpec((HC, HC), lambda c, i: (0, 0)),
            pl.BlockSpec((1, HC), lambda c, i: (0, 0)),
            pl.BlockSpec((1, HC), lambda c, i: (0, 0)),
            pl.BlockSpec((1, HC), lambda c, i: (0, 0)),
        ],
        out_specs=[
            pl.BlockSpec((RT, HC), lambda c, i: (c * nb + i, 0)),
            pl.BlockSpec((RT, HALF), lambda c, i: (c * nb + i, 0)),
        ],
        out_shape=[
            jax.ShapeDtypeStruct((2 * N, HC), jnp.float32),
            jax.ShapeDtypeStruct((2 * N, HALF), jnp.float32),
        ],
    )


_prep_nr = _make_prep(False)
_prep_r = _make_prep(True)


def _edge_body(g_hbm, arr_hbm, src_hbm, dst_hbm, zero_hbm, out_hbm,
               src_v, dst_v, srcg_v, dstg_v, rows_v, ar_v, contrib_v,
               acc_sh, sem1, sem2):
    c = lax.axis_index("c")
    s = lax.axis_index("s")
    cn = c * N

    pltpu.sync_copy(zero_hbm, acc_sh.at[pl.ds(s * NPERS, NPERS)])
    plsc.subcore_barrier()

    def do_batch(b):
        eoff = pl.multiple_of(b * B, B)
        pltpu.sync_copy(src_hbm.at[pl.ds(eoff, B)], src_v)
        pltpu.sync_copy(dst_hbm.at[pl.ds(eoff, B)], dst_v)
        for k in range(B // 16):
            sl = pl.ds(k * 16, 16)
            srcg_v[sl] = src_v[sl] + cn
            dstg_v[sl] = dst_v[sl] + cn
        cp1 = pltpu.async_copy(g_hbm.at[srcg_v], rows_v, sem1)
        cp2 = pltpu.async_copy(arr_hbm.at[dstg_v], ar_v, sem2)
        cp1.wait()
        cp2.wait()

        def edge(e, carry):
            for k in range(HALF // 16):
                sl = pl.ds(k * 16, 16)
                sl2 = pl.ds(HALF + k * 16, 16)
                a = rows_v[e, sl] + ar_v[e, sl]
                ew = jnp.exp(jnp.maximum(a, 0.2 * a))
                contrib_v[e, sl] = rows_v[e, sl2] * ew
                contrib_v[e, sl2] = ew
            return carry

        lax.fori_loop(0, B, edge, 0, unroll=2)
        pltpu.sync_copy(contrib_v, acc_sh.at[dst_v], add=True)

    def loop_body(i, carry):
        do_batch(i * NSUB + s)
        return carry

    nfull = NBATCH // NSUB
    lax.fori_loop(0, nfull, loop_body, 0)
    rem = NBATCH - nfull * NSUB

    @pl.when(s < rem)
    def _():
        do_batch(nfull * NSUB + s)

    plsc.subcore_barrier()

    ch = 80

    def fin(j, carry):
        n0 = s * NPERS + j * ch

        @pl.when(n0 < N)
        def _():
            pltpu.sync_copy(acc_sh.at[pl.ds(n0, ch)], rows_v.at[pl.ds(0, ch)])

            def frow(r, carry2):
                for k in range(HALF // 16):
                    sl = pl.ds(k * 16, 16)
                    sl2 = pl.ds(HALF + k * 16, 16)
                    contrib_v[r, sl] = rows_v[r, sl] / (rows_v[r, sl2] + EPS)
                return carry2

            lax.fori_loop(0, ch, frow, 0, unroll=2)
            pltpu.sync_copy(contrib_v.at[pl.ds(0, ch), pl.ds(0, HALF)],
                            out_hbm.at[c, pl.ds(n0, ch), :])
        return carry

    lax.fori_loop(0, NPERS // ch, fin, 0)


_edge_call = pl.kernel(
    _edge_body,
    out_type=jax.ShapeDtypeStruct((2, N, HALF), jnp.float32),
    mesh=plsc.VectorSubcoreMesh(core_axis_name="c", subcore_axis_name="s"),
    compiler_params=pltpu.CompilerParams(use_tc_tiling_on_sc=False),
    scratch_types=[
        pltpu.VMEM((B,), jnp.int32),
        pltpu.VMEM((B,), jnp.int32),
        pltpu.VMEM((B,), jnp.int32),
        pltpu.VMEM((B,), jnp.int32),
        pltpu.VMEM((B, HC), jnp.float32),
        pltpu.VMEM((B, HALF), jnp.float32),
        pltpu.VMEM((B, HC), jnp.float32),
        pltpu.VMEM_SHARED((ACCN, HC), jnp.float32),
        pltpu.SemaphoreType.DMA,
        pltpu.SemaphoreType.DMA,
    ],
)


def _head_body(h0_ref, h1_ref, w1t_ref, b1_ref, w2t_ref, b2_ref, o_ref):
    hb = jnp.maximum(jnp.concatenate([h0_ref[...], h1_ref[...]], axis=1), 0.0)
    t = jnp.dot(hb, w1t_ref[...], preferred_element_type=jnp.float32) + b1_ref[...]
    t = jnp.dot(t, w2t_ref[...], preferred_element_type=jnp.float32) + b2_ref[...]
    m = jnp.max(t, axis=1, keepdims=True)
    u = t - m
    lse = jnp.log(jnp.sum(jnp.exp(u), axis=1, keepdims=True))
    o_ref[...] = u - lse


_head_call = pl.pallas_call(
    _head_body,
    grid=(N // RT,),
    in_specs=[
        pl.BlockSpec((RT, HALF), lambda i: (i, 0)),
        pl.BlockSpec((RT, HALF), lambda i: (i, 0)),
        pl.BlockSpec((HC, HALF), lambda i: (0, 0)),
        pl.BlockSpec((1, HALF), lambda i: (0, 0)),
        pl.BlockSpec((HALF, HC), lambda i: (0, 0)),
        pl.BlockSpec((1, HC), lambda i: (0, 0)),
    ],
    out_specs=pl.BlockSpec((RT, HC), lambda i: (i, 0)),
    out_shape=jax.ShapeDtypeStruct((N, HC), jnp.float32),
)


def kernel(x, edge_index, batch, W1, b1, attl1, attr1, W2, b2, attl2, attr2,
           pW1, pb1, pW2, pb2):
    src = edge_index[0].astype(jnp.int32)
    dst = edge_index[1].astype(jnp.int32)
    zeros = jnp.zeros((NPERS, HC), jnp.float32)

    g1, arr1 = _prep_nr(x[:, :HALF], x[:, HALF:], W1.T, b1[None, :],
                        attl1.reshape(1, HC), attr1.reshape(1, HC))
    h1 = _edge_call(g1, arr1, src, dst, zeros)
    g2, arr2 = _prep_r(h1[0], h1[1], W2.T, b2[None, :],
                       attl2.reshape(1, HC), attr2.reshape(1, HC))
    h2 = _edge_call(g2, arr2, src, dst, zeros)
    return _head_call(h2[0], h2[1], pW1.T, pb1[None, :], pW2.T, pb2[None, :])
